# speedup vs baseline: 2.8602x; 2.8602x over previous
import functools

import jax
import jax.numpy as jnp
from jax.experimental import pallas as pl
from jax.experimental.pallas import tpu as pltpu

NOBJ = 1024
NREL = 4096
OBJ_DIM = 4096
H = 512
NCLS = 151
NRC = 51
NRCP = 128


def _node_proj_body(obj_fmaps, W_obj, b_obj, W_g1a, W_g1b, cls_embp, labels,
                    f_o, fa_o, fb_o, g_o):
    f = jnp.dot(obj_fmaps[...], W_obj[...], preferred_element_type=jnp.float32)
    f = f + b_obj[...]
    f_o[...] = f
    fa_o[...] = jnp.dot(f, W_g1a[...], preferred_element_type=jnp.float32)
    fb_o[...] = jnp.dot(f, W_g1b[...], preferred_element_type=jnp.float32)
    lab = labels[...]
    oh = (lab == jax.lax.broadcasted_iota(jnp.int32, (NOBJ, 256), 1))
    emb = jnp.dot(oh.astype(jnp.float32), cls_embp[...],
                  preferred_element_type=jnp.float32)
    g_o[...] = emb + f


def _fold_body(W_rel, W_g1c, b_rel, b_g1, Wrc_o, crow_o):
    Wrc_o[...] = jnp.dot(W_rel[...], W_g1c[...],
                         preferred_element_type=jnp.float32)
    crow_o[...] = jnp.dot(b_rel[...], W_g1c[...],
                          preferred_element_type=jnp.float32) + b_g1[...]


def _vc_body(vr, Wrc, vc_o):
    vc_o[...] = jnp.dot(vr[...], Wrc[...], preferred_element_type=jnp.float32)


def _box_feats(bsT, boT):
    def row(t, i):
        return t[i:i + 1, :]
    sx1, sy1, sx2, sy2 = (row(bsT, i) for i in range(4))
    ox1, oy1, ox2, oy2 = (row(boT, i) for i in range(4))
    px1 = jnp.minimum(sx1, ox1)
    py1 = jnp.minimum(sy1, oy1)
    px2 = jnp.maximum(sx2, ox2)
    py2 = jnp.maximum(sy2, oy2)

    def ctr(x1, y1, x2, y2):
        return ((x1 + x2) * 0.5, (y1 + y2) * 0.5,
                (x2 - x1) * 0.5, (y2 - y1) * 0.5)

    scx, scy, sw, sh = ctr(sx1, sy1, sx2, sy2)
    ocx, ocy, ow, oh = ctr(ox1, oy1, ox2, oy2)
    pcx, pcy, pw, ph = ctr(px1, py1, px2, py2)

    def delta(a, b):
        (acx, acy, aw, ah), (bcx, bcy, bw, bh) = a, b
        return [(acx - bcx) / bw, (acy - bcy) / bh,
                jnp.log(aw / bw), jnp.log(ah * bh)]

    def c5(x1, y1, x2, y2):
        return [x1 / 592.0, y1 / 592.0, (x1 + x2) / 592.0,
                (y1 + y2) / 592.0, x2 * y2 / (592.0 ** 2)]

    rows = []
    rows += delta((scx, scy, sw, sh), (ocx, ocy, ow, oh))
    rows += delta((scx, scy, sw, sh), (pcx, pcy, pw, ph))
    rows += delta((pcx, pcy, pw, ph), (ocx, ocy, ow, oh))
    rows += c5(sx1, sy1, sx2, sy2)
    rows += c5(ox1, oy1, ox2, oy2)
    return rows


def _edge_e_body(bsT, boT, W_boxp, crow, E1, vc, e_o):
    rows = _box_feats(bsT[...], boT[...])
    bfT = jnp.concatenate(rows + [jnp.zeros_like(rows[0])] * 10, axis=0)
    bfW = jax.lax.dot_general(bfT, W_boxp[...], (((0,), (0,)), ((), ())),
                              preferred_element_type=jnp.float32)
    e_o[...] = jax.nn.relu(E1[...] + vc[...] + bfW + crow[...])


def _gather_pair_body(idx_a, idx_b, tab_a, tab_b, out_o):
    ia = idx_a[...]
    ib = idx_b[...]
    ids = jax.lax.broadcasted_iota(jnp.int32, (ia.shape[0], NOBJ), 1)
    oha = (ia == ids).astype(jnp.float32)
    ohb = (ib == ids).astype(jnp.float32)
    out_o[...] = (jnp.dot(oha, tab_a[...], preferred_element_type=jnp.float32)
                  + jnp.dot(ohb, tab_b[...], preferred_element_type=jnp.float32))


def _segsum_body(sub_row, obj_row, e, agg_o, *, bn):
    n0 = pl.program_id(0) * bn
    ids = jax.lax.broadcasted_iota(jnp.int32, (bn, NREL), 0) + n0
    s = sub_row[0:1, :]
    o = obj_row[0:1, :]
    pt = (ids == s).astype(jnp.float32) + (ids == o).astype(jnp.float32)
    agg_o[...] = jnp.dot(pt, e[...], preferred_element_type=jnp.float32)


def _node_mm_body(agg, W, out_o):
    out_o[...] = jax.nn.relu(
        jnp.dot(agg[...], W[...], preferred_element_type=jnp.float32))


def _edge_update_body(idx_a, idx_b, node, e, W_outp, e2_o, l_o):
    ia = idx_a[...]
    ib = idx_b[...]
    ids = jax.lax.broadcasted_iota(jnp.int32, (ia.shape[0], NOBJ), 1)
    p = ((ia == ids).astype(jnp.float32) + (ib == ids).astype(jnp.float32))
    np_ = jnp.dot(p, node[...], preferred_element_type=jnp.float32)
    e2 = jax.nn.relu(e[...] + np_)
    e2_o[...] = e2
    l_o[...] = jnp.dot(e2, W_outp[...], preferred_element_type=jnp.float32)


def _hh_body(idx_a, idx_b, g, bpT, W_bp, W_voutp, l1, lv_o, rel0_o):
    ia = idx_a[...]
    ib = idx_b[...]
    ids = jax.lax.broadcasted_iota(jnp.int32, (ia.shape[0], NOBJ), 1)
    p = ((ia == ids).astype(jnp.float32) + (ib == ids).astype(jnp.float32))
    gp = jnp.dot(p, g[...], preferred_element_type=jnp.float32)
    bpW = jax.lax.dot_general(bpT[...], W_bp[...], (((0,), (0,)), ((), ())),
                              preferred_element_type=jnp.float32)
    hh = jax.nn.relu(gp + bpW)
    lv = jnp.dot(hh, W_voutp[...], preferred_element_type=jnp.float32)
    lv_o[...] = lv
    rel0_o[...] = l1[...] + lv


def _f32(shape):
    return jax.ShapeDtypeStruct(shape, jnp.float32)


def kernel(obj_fmaps, obj_logits, rel_inds, vr, obj_labels, bboxes,
           obj_logits_fc, W_obj, b_obj, W_rel, b_rel, W_g1, W_box, b_g1,
           W_n1, W_n2, W_out1, W_out2, cls_emb, W_b, W_vout):
    sub = rel_inds[:, 1]
    objn = rel_inds[:, 2]
    sub2d = sub.reshape(NREL, 1)
    obj2d = objn.reshape(NREL, 1)
    sub_row = jnp.broadcast_to(sub[None, :], (8, NREL))
    obj_row = jnp.broadcast_to(objn[None, :], (8, NREL))
    lab2d = obj_labels.reshape(NOBJ, 1)

    W_g1a = W_g1[:H]
    W_g1b = W_g1[H:2 * H]
    W_g1c = W_g1[2 * H:]
    cls_embp = jnp.zeros((256, H), jnp.float32).at[:NCLS].set(cls_emb)
    W_boxp = jnp.zeros((32, H), jnp.float32).at[:22].set(W_box)
    W_out1p = jnp.zeros((H, NRCP), jnp.float32).at[:, :NRC].set(W_out1)
    W_out2p = jnp.zeros((H, NRCP), jnp.float32).at[:, :NRC].set(W_out2)
    W_voutp = jnp.zeros((H, NRCP), jnp.float32).at[:, :NRC].set(W_vout)
    W_bp = jnp.zeros((16, H), jnp.float32).at[:8].set(W_b)
    b_obj_r = b_obj.reshape(1, H)
    b_rel_r = b_rel.reshape(1, H)
    b_g1_r = b_g1.reshape(1, H)

    f, fa, fb, g = pl.pallas_call(
        _node_proj_body,
        out_shape=(_f32((NOBJ, H)),) * 4,
    )(obj_fmaps, W_obj, b_obj_r, W_g1a, W_g1b, cls_embp, lab2d)

    Wrc, crow = pl.pallas_call(
        _fold_body,
        out_shape=(_f32((OBJ_DIM, H)), _f32((1, H))),
    )(W_rel, W_g1c, b_rel_r, b_g1_r)

    BM = 512
    vc = pl.pallas_call(
        _vc_body,
        grid=(NREL // BM,),
        in_specs=[pl.BlockSpec((BM, OBJ_DIM), lambda i: (i, 0)),
                  pl.BlockSpec((OBJ_DIM, H), lambda i: (0, 0))],
        out_specs=pl.BlockSpec((BM, H), lambda i: (i, 0)),
        out_shape=_f32((NREL, H)),
    )(vr, Wrc)

    BE = 1024
    E1 = pl.pallas_call(
        _gather_pair_body,
        grid=(NREL // BE,),
        in_specs=[pl.BlockSpec((BE, 1), lambda i: (i, 0)),
                  pl.BlockSpec((BE, 1), lambda i: (i, 0)),
                  pl.BlockSpec((NOBJ, H), lambda i: (0, 0)),
                  pl.BlockSpec((NOBJ, H), lambda i: (0, 0))],
        out_specs=pl.BlockSpec((BE, H), lambda i: (i, 0)),
        out_shape=_f32((NREL, H)),
    )(sub2d, obj2d, fa, fb)

    bs_T = bboxes.T[:, sub]
    bs_T = jnp.concatenate([bs_T, jnp.zeros((4, NREL), jnp.float32)], 0)
    bo_T = bboxes.T[:, objn]
    bo_T = jnp.concatenate([bo_T, jnp.zeros((4, NREL), jnp.float32)], 0)

    e = pl.pallas_call(
        _edge_e_body,
        grid=(NREL // BE,),
        in_specs=[pl.BlockSpec((8, BE), lambda i: (0, i)),
                  pl.BlockSpec((8, BE), lambda i: (0, i)),
                  pl.BlockSpec((32, H), lambda i: (0, 0)),
                  pl.BlockSpec((1, H), lambda i: (0, 0)),
                  pl.BlockSpec((BE, H), lambda i: (i, 0)),
                  pl.BlockSpec((BE, H), lambda i: (i, 0))],
        out_specs=pl.BlockSpec((BE, H), lambda i: (i, 0)),
        out_shape=_f32((NREL, H)),
    )(bs_T, bo_T, W_boxp, crow, E1, vc)

    BN = 512
    seg = pl.pallas_call(
        functools.partial(_segsum_body, bn=BN),
        grid=(NOBJ // BN,),
        in_specs=[pl.BlockSpec((8, NREL), lambda i: (0, 0)),
                  pl.BlockSpec((8, NREL), lambda i: (0, 0)),
                  pl.BlockSpec((NREL, H), lambda i: (0, 0))],
        out_specs=pl.BlockSpec((BN, H), lambda i: (i, 0)),
        out_shape=_f32((NOBJ, H)),
    )
    agg = seg(sub_row, obj_row, e)

    node_mm = pl.pallas_call(
        _node_mm_body,
        out_shape=_f32((NOBJ, H)),
    )
    node = node_mm(agg, W_n1)

    edge_update = pl.pallas_call(
        _edge_update_body,
        grid=(NREL // BE,),
        in_specs=[pl.BlockSpec((BE, 1), lambda i: (i, 0)),
                  pl.BlockSpec((BE, 1), lambda i: (i, 0)),
                  pl.BlockSpec((NOBJ, H), lambda i: (0, 0)),
                  pl.BlockSpec((BE, H), lambda i: (i, 0)),
                  pl.BlockSpec((H, NRCP), lambda i: (0, 0))],
        out_specs=(pl.BlockSpec((BE, H), lambda i: (i, 0)),
                   pl.BlockSpec((BE, NRCP), lambda i: (i, 0))),
        out_shape=(_f32((NREL, H)), _f32((NREL, NRCP))),
    )
    e2, l1p = edge_update(sub2d, obj2d, node, e, W_out1p)

    agg2 = seg(sub_row, obj_row, e2)
    node2 = node_mm(agg2, W_n2)
    _, l2p = edge_update(sub2d, obj2d, node2, e2, W_out2p)

    bpT = jnp.concatenate([bs_T[:4], bo_T[:4],
                           jnp.zeros((8, NREL), jnp.float32)], 0) / 592.0
    lvp, rel0p = pl.pallas_call(
        _hh_body,
        grid=(NREL // BE,),
        in_specs=[pl.BlockSpec((BE, 1), lambda i: (i, 0)),
                  pl.BlockSpec((BE, 1), lambda i: (i, 0)),
                  pl.BlockSpec((NOBJ, H), lambda i: (0, 0)),
                  pl.BlockSpec((16, BE), lambda i: (0, i)),
                  pl.BlockSpec((16, H), lambda i: (0, 0)),
                  pl.BlockSpec((H, NRCP), lambda i: (0, 0)),
                  pl.BlockSpec((BE, NRCP), lambda i: (i, 0))],
        out_specs=(pl.BlockSpec((BE, NRCP), lambda i: (i, 0)),
                   pl.BlockSpec((BE, NRCP), lambda i: (i, 0))),
        out_shape=(_f32((NREL, NRCP)), _f32((NREL, NRCP))),
    )(sub2d, obj2d, g, bpT, W_bp, W_voutp, l1p)

    rel0 = rel0p[:, :NRC]
    l2 = l2p[:, :NRC]
    lv = lvp[:, :NRC]
    return (obj_logits, obj_labels, rel0, l2, lv)


# speedup vs baseline: 2.8667x; 1.0023x over previous
import functools

import jax
import jax.numpy as jnp
from jax import lax
from jax.experimental import pallas as pl
from jax.experimental.pallas import tpu as pltpu
from jax.experimental.pallas import tpu_sc as plsc

NOBJ = 1024
NREL = 4096
OBJ_DIM = 4096
H = 512
NCLS = 151
NRC = 51
NRCP = 128


def _node_proj_body(obj_fmaps, W_obj, b_obj, W_g1a, W_g1b, cls_embp, labels,
                    f_o, fa_o, fb_o, g_o):
    f = jnp.dot(obj_fmaps[...], W_obj[...], preferred_element_type=jnp.float32)
    f = f + b_obj[...]
    f_o[...] = f
    fa_o[...] = jnp.dot(f, W_g1a[...], preferred_element_type=jnp.float32)
    fb_o[...] = jnp.dot(f, W_g1b[...], preferred_element_type=jnp.float32)
    lab = labels[...]
    oh = (lab == jax.lax.broadcasted_iota(jnp.int32, (NOBJ, 256), 1))
    emb = jnp.dot(oh.astype(jnp.float32), cls_embp[...],
                  preferred_element_type=jnp.float32)
    g_o[...] = emb + f


def _fold_body(W_rel, W_g1c, b_rel, b_g1, Wrc_o, crow_o):
    Wrc_o[...] = jnp.dot(W_rel[...], W_g1c[...],
                         preferred_element_type=jnp.float32)
    crow_o[...] = jnp.dot(b_rel[...], W_g1c[...],
                          preferred_element_type=jnp.float32) + b_g1[...]


def _vc_body(vr, Wrc, vc_o):
    vc_o[...] = jnp.dot(vr[...], Wrc[...], preferred_element_type=jnp.float32)


def _box_feats(bsT, boT):
    def row(t, i):
        return t[i:i + 1, :]
    sx1, sy1, sx2, sy2 = (row(bsT, i) for i in range(4))
    ox1, oy1, ox2, oy2 = (row(boT, i) for i in range(4))
    px1 = jnp.minimum(sx1, ox1)
    py1 = jnp.minimum(sy1, oy1)
    px2 = jnp.maximum(sx2, ox2)
    py2 = jnp.maximum(sy2, oy2)

    def ctr(x1, y1, x2, y2):
        return ((x1 + x2) * 0.5, (y1 + y2) * 0.5,
                (x2 - x1) * 0.5, (y2 - y1) * 0.5)

    scx, scy, sw, sh = ctr(sx1, sy1, sx2, sy2)
    ocx, ocy, ow, oh = ctr(ox1, oy1, ox2, oy2)
    pcx, pcy, pw, ph = ctr(px1, py1, px2, py2)

    def delta(a, b):
        (acx, acy, aw, ah), (bcx, bcy, bw, bh) = a, b
        return [(acx - bcx) / bw, (acy - bcy) / bh,
                jnp.log(aw / bw), jnp.log(ah * bh)]

    def c5(x1, y1, x2, y2):
        return [x1 / 592.0, y1 / 592.0, (x1 + x2) / 592.0,
                (y1 + y2) / 592.0, x2 * y2 / (592.0 ** 2)]

    rows = []
    rows += delta((scx, scy, sw, sh), (ocx, ocy, ow, oh))
    rows += delta((scx, scy, sw, sh), (pcx, pcy, pw, ph))
    rows += delta((pcx, pcy, pw, ph), (ocx, ocy, ow, oh))
    rows += c5(sx1, sy1, sx2, sy2)
    rows += c5(ox1, oy1, ox2, oy2)
    return rows


def _edge_e_body(bsT, boT, W_boxp, crow, E1, vc, e_o):
    rows = _box_feats(bsT[...], boT[...])
    bfT = jnp.concatenate(rows + [jnp.zeros_like(rows[0])] * 10, axis=0)
    bfW = jax.lax.dot_general(bfT, W_boxp[...], (((0,), (0,)), ((), ())),
                              preferred_element_type=jnp.float32)
    e_o[...] = jax.nn.relu(E1[...] + vc[...] + bfW + crow[...])


_SC_NC = 2
_SC_NS = 16
_SC_NW = _SC_NC * _SC_NS
_SC_CHUNK = 32


def _sc_gather_pair(tab_a, tab_b, idx_a, idx_b):
    n, d = NREL, H
    bpw = n // _SC_NW
    nch = bpw // _SC_CHUNK
    mesh = plsc.VectorSubcoreMesh(core_axis_name="c", subcore_axis_name="s")

    @functools.partial(
        pl.kernel, mesh=mesh,
        out_type=jax.ShapeDtypeStruct((n, d), jnp.float32),
        scratch_types=[
            pltpu.VMEM((_SC_CHUNK,), jnp.int32),
            pltpu.VMEM((_SC_CHUNK,), jnp.int32),
            pltpu.VMEM((_SC_CHUNK, d), jnp.float32),
            pltpu.VMEM((_SC_CHUNK, d), jnp.float32),
            pltpu.SemaphoreType.DMA,
            pltpu.SemaphoreType.DMA,
        ],
    )
    def k(tab_a_h, tab_b_h, idx_a_h, idx_b_h, out_h, ia_v, ib_v, ra_v, rb_v,
          sa, sb):
        wid = lax.axis_index("s") * _SC_NC + lax.axis_index("c")
        base = wid * bpw

        def chunk(c, _):
            off = base + c * _SC_CHUNK
            pltpu.sync_copy(idx_a_h.at[pl.ds(off, _SC_CHUNK)], ia_v)
            pltpu.sync_copy(idx_b_h.at[pl.ds(off, _SC_CHUNK)], ib_v)
            cpa = pltpu.async_copy(tab_a_h.at[ia_v], ra_v, sa)
            cpb = pltpu.async_copy(tab_b_h.at[ib_v], rb_v, sb)
            cpa.wait()
            cpb.wait()

            def addrow(r, _):
                for j in range(d // 16):
                    ra_v[r, pl.ds(j * 16, 16)] = (
                        ra_v[r, pl.ds(j * 16, 16)] + rb_v[r, pl.ds(j * 16, 16)])
                return 0

            lax.fori_loop(0, _SC_CHUNK, addrow, 0)
            pltpu.sync_copy(ra_v, out_h.at[pl.ds(off, _SC_CHUNK)])
            return 0

        lax.fori_loop(0, nch, chunk, 0)

    return k(tab_a, tab_b, idx_a, idx_b)


def _gather_pair_body(idx_a, idx_b, tab_a, tab_b, out_o):
    ia = idx_a[...]
    ib = idx_b[...]
    ids = jax.lax.broadcasted_iota(jnp.int32, (ia.shape[0], NOBJ), 1)
    oha = (ia == ids).astype(jnp.float32)
    ohb = (ib == ids).astype(jnp.float32)
    out_o[...] = (jnp.dot(oha, tab_a[...], preferred_element_type=jnp.float32)
                  + jnp.dot(ohb, tab_b[...], preferred_element_type=jnp.float32))


def _segsum_body(sub_row, obj_row, e, agg_o, *, bn):
    n0 = pl.program_id(0) * bn
    ids = jax.lax.broadcasted_iota(jnp.int32, (bn, NREL), 0) + n0
    s = sub_row[0:1, :]
    o = obj_row[0:1, :]
    pt = (ids == s).astype(jnp.float32) + (ids == o).astype(jnp.float32)
    agg_o[...] = jnp.dot(pt, e[...], preferred_element_type=jnp.float32)


def _node_mm_body(agg, W, out_o):
    out_o[...] = jax.nn.relu(
        jnp.dot(agg[...], W[...], preferred_element_type=jnp.float32))


def _edge_update_body(idx_a, idx_b, node, e, W_outp, e2_o, l_o):
    ia = idx_a[...]
    ib = idx_b[...]
    ids = jax.lax.broadcasted_iota(jnp.int32, (ia.shape[0], NOBJ), 1)
    p = ((ia == ids).astype(jnp.float32) + (ib == ids).astype(jnp.float32))
    np_ = jnp.dot(p, node[...], preferred_element_type=jnp.float32)
    e2 = jax.nn.relu(e[...] + np_)
    e2_o[...] = e2
    l_o[...] = jnp.dot(e2, W_outp[...], preferred_element_type=jnp.float32)


def _hh_body(idx_a, idx_b, g, bpT, W_bp, W_voutp, l1, lv_o, rel0_o):
    ia = idx_a[...]
    ib = idx_b[...]
    ids = jax.lax.broadcasted_iota(jnp.int32, (ia.shape[0], NOBJ), 1)
    p = ((ia == ids).astype(jnp.float32) + (ib == ids).astype(jnp.float32))
    gp = jnp.dot(p, g[...], preferred_element_type=jnp.float32)
    bpW = jax.lax.dot_general(bpT[...], W_bp[...], (((0,), (0,)), ((), ())),
                              preferred_element_type=jnp.float32)
    hh = jax.nn.relu(gp + bpW)
    lv = jnp.dot(hh, W_voutp[...], preferred_element_type=jnp.float32)
    lv_o[...] = lv
    rel0_o[...] = l1[...] + lv


def _f32(shape):
    return jax.ShapeDtypeStruct(shape, jnp.float32)


def kernel(obj_fmaps, obj_logits, rel_inds, vr, obj_labels, bboxes,
           obj_logits_fc, W_obj, b_obj, W_rel, b_rel, W_g1, W_box, b_g1,
           W_n1, W_n2, W_out1, W_out2, cls_emb, W_b, W_vout):
    sub = rel_inds[:, 1]
    objn = rel_inds[:, 2]
    sub2d = sub.reshape(NREL, 1)
    obj2d = objn.reshape(NREL, 1)
    sub_row = jnp.broadcast_to(sub[None, :], (8, NREL))
    obj_row = jnp.broadcast_to(objn[None, :], (8, NREL))
    lab2d = obj_labels.reshape(NOBJ, 1)

    W_g1a = W_g1[:H]
    W_g1b = W_g1[H:2 * H]
    W_g1c = W_g1[2 * H:]
    cls_embp = jnp.zeros((256, H), jnp.float32).at[:NCLS].set(cls_emb)
    W_boxp = jnp.zeros((32, H), jnp.float32).at[:22].set(W_box)
    W_out1p = jnp.zeros((H, NRCP), jnp.float32).at[:, :NRC].set(W_out1)
    W_out2p = jnp.zeros((H, NRCP), jnp.float32).at[:, :NRC].set(W_out2)
    W_voutp = jnp.zeros((H, NRCP), jnp.float32).at[:, :NRC].set(W_vout)
    W_bp = jnp.zeros((16, H), jnp.float32).at[:8].set(W_b)
    b_obj_r = b_obj.reshape(1, H)
    b_rel_r = b_rel.reshape(1, H)
    b_g1_r = b_g1.reshape(1, H)

    f, fa, fb, g = pl.pallas_call(
        _node_proj_body,
        out_shape=(_f32((NOBJ, H)),) * 4,
    )(obj_fmaps, W_obj, b_obj_r, W_g1a, W_g1b, cls_embp, lab2d)

    Wrc, crow = pl.pallas_call(
        _fold_body,
        out_shape=(_f32((OBJ_DIM, H)), _f32((1, H))),
    )(W_rel, W_g1c, b_rel_r, b_g1_r)

    BM = 512
    vc = pl.pallas_call(
        _vc_body,
        grid=(NREL // BM,),
        in_specs=[pl.BlockSpec((BM, OBJ_DIM), lambda i: (i, 0)),
                  pl.BlockSpec((OBJ_DIM, H), lambda i: (0, 0))],
        out_specs=pl.BlockSpec((BM, H), lambda i: (i, 0)),
        out_shape=_f32((NREL, H)),
    )(vr, Wrc)

    BE = 1024
    E1 = _sc_gather_pair(fa, fb, sub, objn)

    bs_T = bboxes.T[:, sub]
    bs_T = jnp.concatenate([bs_T, jnp.zeros((4, NREL), jnp.float32)], 0)
    bo_T = bboxes.T[:, objn]
    bo_T = jnp.concatenate([bo_T, jnp.zeros((4, NREL), jnp.float32)], 0)

    e = pl.pallas_call(
        _edge_e_body,
        grid=(NREL // BE,),
        in_specs=[pl.BlockSpec((8, BE), lambda i: (0, i)),
                  pl.BlockSpec((8, BE), lambda i: (0, i)),
                  pl.BlockSpec((32, H), lambda i: (0, 0)),
                  pl.BlockSpec((1, H), lambda i: (0, 0)),
                  pl.BlockSpec((BE, H), lambda i: (i, 0)),
                  pl.BlockSpec((BE, H), lambda i: (i, 0))],
        out_specs=pl.BlockSpec((BE, H), lambda i: (i, 0)),
        out_shape=_f32((NREL, H)),
    )(bs_T, bo_T, W_boxp, crow, E1, vc)

    BN = 512
    seg = pl.pallas_call(
        functools.partial(_segsum_body, bn=BN),
        grid=(NOBJ // BN,),
        in_specs=[pl.BlockSpec((8, NREL), lambda i: (0, 0)),
                  pl.BlockSpec((8, NREL), lambda i: (0, 0)),
                  pl.BlockSpec((NREL, H), lambda i: (0, 0))],
        out_specs=pl.BlockSpec((BN, H), lambda i: (i, 0)),
        out_shape=_f32((NOBJ, H)),
    )
    agg = seg(sub_row, obj_row, e)

    node_mm = pl.pallas_call(
        _node_mm_body,
        out_shape=_f32((NOBJ, H)),
    )
    node = node_mm(agg, W_n1)

    edge_update = pl.pallas_call(
        _edge_update_body,
        grid=(NREL // BE,),
        in_specs=[pl.BlockSpec((BE, 1), lambda i: (i, 0)),
                  pl.BlockSpec((BE, 1), lambda i: (i, 0)),
                  pl.BlockSpec((NOBJ, H), lambda i: (0, 0)),
                  pl.BlockSpec((BE, H), lambda i: (i, 0)),
                  pl.BlockSpec((H, NRCP), lambda i: (0, 0))],
        out_specs=(pl.BlockSpec((BE, H), lambda i: (i, 0)),
                   pl.BlockSpec((BE, NRCP), lambda i: (i, 0))),
        out_shape=(_f32((NREL, H)), _f32((NREL, NRCP))),
    )
    e2, l1p = edge_update(sub2d, obj2d, node, e, W_out1p)

    agg2 = seg(sub_row, obj_row, e2)
    node2 = node_mm(agg2, W_n2)
    _, l2p = edge_update(sub2d, obj2d, node2, e2, W_out2p)

    bpT = jnp.concatenate([bs_T[:4], bo_T[:4],
                           jnp.zeros((8, NREL), jnp.float32)], 0) / 592.0
    lvp, rel0p = pl.pallas_call(
        _hh_body,
        grid=(NREL // BE,),
        in_specs=[pl.BlockSpec((BE, 1), lambda i: (i, 0)),
                  pl.BlockSpec((BE, 1), lambda i: (i, 0)),
                  pl.BlockSpec((NOBJ, H), lambda i: (0, 0)),
                  pl.BlockSpec((16, BE), lambda i: (0, i)),
                  pl.BlockSpec((16, H), lambda i: (0, 0)),
                  pl.BlockSpec((H, NRCP), lambda i: (0, 0)),
                  pl.BlockSpec((BE, NRCP), lambda i: (i, 0))],
        out_specs=(pl.BlockSpec((BE, NRCP), lambda i: (i, 0)),
                   pl.BlockSpec((BE, NRCP), lambda i: (i, 0))),
        out_shape=(_f32((NREL, NRCP)), _f32((NREL, NRCP))),
    )(sub2d, obj2d, g, bpT, W_bp, W_voutp, l1p)

    rel0 = rel0p[:, :NRC]
    l2 = l2p[:, :NRC]
    lv = lvp[:, :NRC]
    return (obj_logits, obj_labels, rel0, l2, lv)
